# baseline (speedup 1.0000x reference)
import functools

import jax
import jax.numpy as jnp
from jax import lax
from jax.experimental import pallas as pl
from jax.experimental.pallas import tpu as pltpu
from jax.experimental.pallas import tpu_sc as plsc

EMB = 64
SEQ = 200
NC = 2
NS = 16
NW = NC * NS
LANES = 16
TOKENS = 1024 * SEQ
TOK_PER_W = TOKENS // NW
CHUNK = 640
N_CHUNKS = TOK_PER_W // CHUNK
IDX_SUB = 128
N_SUB = CHUNK // IDX_SUB
GROUPS = CHUNK // LANES
EPS = 1e-12


def _rsqrt_newton(x):
    i = plsc.bitcast(x, jnp.int32)
    i = jnp.int32(0x5F3759DF) - lax.shift_right_logical(i, 1)
    y = plsc.bitcast(i, jnp.float32)
    for _ in range(3):
        y = y * (1.5 - 0.5 * x * y * y)
    return y


def _sc_kernel(ids_hbm, word_hbm, type_hbm, pos_hbm, gamma_hbm, beta_hbm,
               out_hbm, idx_v, rows_v, bias_v, type_v, gamma_v, beta_v, sem):
    wid = lax.axis_index("s") * NC + lax.axis_index("c")
    iota = lax.iota(jnp.int32, LANES)

    pltpu.sync_copy(pos_hbm.at[pl.ds(0, SEQ)], bias_v)
    pltpu.sync_copy(type_hbm.at[0], type_v)
    pltpu.sync_copy(gamma_hbm, gamma_v)
    pltpu.sync_copy(beta_hbm, beta_v)

    def add_t0(r, carry):
        for c in range(EMB // LANES):
            sl = pl.ds(c * LANES, LANES)
            bias_v[r, sl] = bias_v[r, sl] + type_v[sl]
        return carry

    lax.fori_loop(0, SEQ, add_t0, 0)

    def chunk_body(ci, carry):
        tok0 = wid * TOK_PER_W + ci * CHUNK
        pltpu.sync_copy(ids_hbm.at[pl.ds(tok0, CHUNK)], idx_v)
        copies = [
            pltpu.async_copy(
                word_hbm.at[idx_v.at[pl.ds(k * IDX_SUB, IDX_SUB)]],
                rows_v.at[pl.ds(k * IDX_SUB, IDX_SUB)],
                sem,
            )
            for k in range(N_SUB)
        ]
        for c in copies:
            c.wait()

        def group_body(g, gcarry):
            tok_vec = g * LANES + iota
            pos_vec = lax.rem(tok0 + tok_vec, SEQ)
            acc = jnp.zeros((LANES,), jnp.float32)
            acc2 = jnp.zeros((LANES,), jnp.float32)
            for j in range(EMB):
                col = jnp.full((LANES,), j, jnp.int32)
                v = plsc.load_gather(rows_v, [tok_vec, col])
                b = plsc.load_gather(bias_v, [pos_vec, col])
                s = v + b
                plsc.store_scatter(rows_v, [tok_vec, col], s)
                acc = acc + s
                acc2 = acc2 + s * s
            mean = acc * (1.0 / EMB)
            var = acc2 * (1.0 / EMB) - mean * mean
            rstd = _rsqrt_newton(var + EPS)
            for j in range(EMB):
                col = jnp.full((LANES,), j, jnp.int32)
                s = plsc.load_gather(rows_v, [tok_vec, col])
                gj = plsc.load_gather(gamma_v, [col])
                bj = plsc.load_gather(beta_v, [col])
                y = (s - mean) * rstd * gj + bj
                plsc.store_scatter(rows_v, [tok_vec, col], y)
            return gcarry

        lax.fori_loop(0, GROUPS, group_body, 0)
        pltpu.sync_copy(rows_v, out_hbm.at[pl.ds(tok0, CHUNK)])
        return carry

    lax.fori_loop(0, N_CHUNKS, chunk_body, 0)


def kernel(input_ids, word_emb, type_emb, pos_emb, gamma, beta):
    batch, seq = input_ids.shape
    ids1d = input_ids.reshape(TOKENS).astype(jnp.int32)
    mesh = plsc.VectorSubcoreMesh(core_axis_name="c", subcore_axis_name="s")
    run = pl.kernel(
        _sc_kernel,
        out_type=jax.ShapeDtypeStruct((TOKENS, EMB), jnp.float32),
        mesh=mesh,
        compiler_params=pltpu.CompilerParams(
            needs_layout_passes=False, use_tc_tiling_on_sc=False
        ),
        scratch_types=[
            pltpu.VMEM((CHUNK,), jnp.int32),
            pltpu.VMEM((CHUNK, EMB), jnp.float32),
            pltpu.VMEM((SEQ, EMB), jnp.float32),
            pltpu.VMEM((EMB,), jnp.float32),
            pltpu.VMEM((EMB,), jnp.float32),
            pltpu.VMEM((EMB,), jnp.float32),
            pltpu.SemaphoreType.DMA,
        ],
    )
    out = run(ids1d, word_emb, type_emb, pos_emb, gamma, beta)
    return out.reshape(batch, seq, EMB)


# speedup vs baseline: 1.8833x; 1.8833x over previous
import functools

import jax
import jax.numpy as jnp
from jax import lax
from jax.experimental import pallas as pl
from jax.experimental.pallas import tpu as pltpu
from jax.experimental.pallas import tpu_sc as plsc

EMB = 64
SEQ = 200
NC = 2
NS = 16
NW = NC * NS
LANES = 16
TOKENS = 1024 * SEQ
TOK_PER_W = TOKENS // NW
CHUNK = 320
N_CHUNKS = TOK_PER_W // CHUNK
IDX_SUB = 80
N_SUB = CHUNK // IDX_SUB
GROUPS = CHUNK // LANES
NACC = 8
BIAS_ROWS = 104
TYPE_ROWS = 2
EPS = 1e-12


def _rsqrt_newton(x):
    i = plsc.bitcast(x, jnp.int32)
    i = jnp.int32(0x5F3759DF) - lax.shift_right_logical(i, 1)
    y = plsc.bitcast(i, jnp.float32)
    for _ in range(3):
        y = y * (1.5 - 0.5 * x * y * y)
    return y


def _sc_kernel(ids_hbm, word_hbm, type_hbm, pos_hbm, gamma_hbm, beta_hbm,
               out_hbm, idx_v, pair_v, rows_v, out_v, bias_v, type_s,
               gamma_v, beta_v, sem):
    wid = lax.axis_index("s") * NC + lax.axis_index("c")
    iota = lax.iota(jnp.int32, LANES)

    pltpu.sync_copy(pos_hbm.at[pl.ds(0, BIAS_ROWS)], bias_v)
    pltpu.sync_copy(type_hbm, type_s)
    pltpu.sync_copy(gamma_hbm, gamma_v)
    pltpu.sync_copy(beta_hbm, beta_v)

    @plsc.parallel_loop(0, SEQ // 2, 1, unroll=2)
    def add_t0(r):
        for c in range(2 * EMB // LANES):
            sl = pl.ds(c * LANES, LANES)
            tsl = pl.ds((c % (EMB // LANES)) * LANES, LANES)
            bias_v[r, sl] = bias_v[r, sl] + type_s[0, tsl]

    def chunk_body(ci, carry):
        tok0 = wid * TOK_PER_W + ci * CHUNK
        pltpu.sync_copy(ids_hbm.at[pl.ds(tok0, CHUNK)], idx_v)

        @plsc.parallel_loop(0, CHUNK // LANES, 1, unroll=2)
        def shift_ids(i):
            sl = pl.ds(i * LANES, LANES)
            pair_v[sl] = lax.shift_right_logical(idx_v[sl], 1)

        copies = [
            pltpu.async_copy(
                word_hbm.at[pair_v.at[pl.ds(k * IDX_SUB, IDX_SUB)]],
                rows_v.at[pl.ds(k * IDX_SUB, IDX_SUB)],
                sem,
            )
            for k in range(N_SUB)
        ]
        for c in copies:
            c.wait()

        @plsc.parallel_loop(0, GROUPS, 1, unroll=1)
        def group_body(g):
            tok_vec = g * LANES + iota
            pos_vec = lax.rem(tok0 + tok_vec, SEQ)
            ids_vec = idx_v[pl.ds(g * LANES, LANES)]
            half = lax.shift_left(
                lax.bitwise_and(ids_vec, jnp.int32(1)), 6)
            brow = lax.shift_right_logical(pos_vec, 1)
            bhalf = lax.shift_left(
                lax.bitwise_and(pos_vec, jnp.int32(1)), 6)
            accs = [jnp.zeros((LANES,), jnp.float32) for _ in range(NACC)]
            acc2s = [jnp.zeros((LANES,), jnp.float32) for _ in range(NACC)]
            for j in range(EMB):
                dcol = lax.bitwise_and(j + iota, jnp.int32(EMB - 1))
                v = plsc.load_gather(rows_v, [tok_vec, half + dcol])
                b = plsc.load_gather(bias_v, [brow, bhalf + dcol])
                s = v + b
                k = j % NACC
                accs[k] = accs[k] + s
                acc2s[k] = acc2s[k] + s * s
            while len(accs) > 1:
                accs = [a + b for a, b in zip(accs[::2], accs[1::2])]
                acc2s = [a + b for a, b in zip(acc2s[::2], acc2s[1::2])]
            mean = accs[0] * (1.0 / EMB)
            var = acc2s[0] * (1.0 / EMB) - mean * mean
            rstd = _rsqrt_newton(var + EPS)
            gvecs = [gamma_v[pl.ds(c * LANES, LANES)] for c in range(EMB // LANES)]
            bvecs = [beta_v[pl.ds(c * LANES, LANES)] for c in range(EMB // LANES)]
            for r in range(LANES):
                m_b = mean[r]
                rs_b = rstd[r]
                hoff = half[r]
                tok_r = g * LANES + r
                pos_r = tok0 + tok_r
                pos_r = lax.rem(pos_r, SEQ)
                boff = lax.shift_left(lax.bitwise_and(pos_r, jnp.int32(1)), 6)
                brow_r = lax.shift_right_logical(pos_r, 1)
                orow = g * (LANES // 2) + (r // 2)
                ocol0 = (r % 2) * EMB
                for c in range(EMB // LANES):
                    s = (rows_v[tok_r, pl.ds(hoff + c * LANES, LANES)]
                         + bias_v[brow_r, pl.ds(boff + c * LANES, LANES)])
                    y = (s - m_b) * rs_b * gvecs[c] + bvecs[c]
                    out_v[orow, pl.ds(ocol0 + c * LANES, LANES)] = y
            return None

        orow0 = pl.multiple_of(tok0 // 2, 8)
        pltpu.sync_copy(out_v, out_hbm.at[pl.ds(orow0, CHUNK // 2)])
        return carry

    lax.fori_loop(0, N_CHUNKS, chunk_body, 0)


def kernel(input_ids, word_emb, type_emb, pos_emb, gamma, beta):
    batch, seq = input_ids.shape
    ids1d = input_ids.reshape(TOKENS).astype(jnp.int32)
    word2 = word_emb.reshape(word_emb.shape[0] // 2, 2 * EMB)
    pos2 = pos_emb.reshape(pos_emb.shape[0] // 2, 2 * EMB)
    mesh = plsc.VectorSubcoreMesh(core_axis_name="c", subcore_axis_name="s")
    run = pl.kernel(
        _sc_kernel,
        out_type=jax.ShapeDtypeStruct((TOKENS // 2, 2 * EMB), jnp.float32),
        mesh=mesh,
        compiler_params=pltpu.CompilerParams(needs_layout_passes=False),
        scratch_types=[
            pltpu.VMEM((CHUNK,), jnp.int32),
            pltpu.VMEM((CHUNK,), jnp.int32),
            pltpu.VMEM((CHUNK, 2 * EMB), jnp.float32),
            pltpu.VMEM((CHUNK // 2, 2 * EMB), jnp.float32),
            pltpu.VMEM((BIAS_ROWS, 2 * EMB), jnp.float32),
            pltpu.VMEM((TYPE_ROWS, EMB), jnp.float32),
            pltpu.VMEM((EMB,), jnp.float32),
            pltpu.VMEM((EMB,), jnp.float32),
            pltpu.SemaphoreType.DMA,
        ],
    )
    out = run(ids1d, word2, type_emb, pos2, gamma, beta)
    return out.reshape(batch, seq, EMB)


# speedup vs baseline: 1.9520x; 1.0365x over previous
import functools

import jax
import jax.numpy as jnp
from jax import lax
from jax.experimental import pallas as pl
from jax.experimental.pallas import tpu as pltpu
from jax.experimental.pallas import tpu_sc as plsc

EMB = 64
SEQ = 200
BATCH = 1024
NC = 2
NS = 16
NW = NC * NS
LANES = 16
SEQ_PER_W = BATCH // NW
SEQ_PER_CHUNK = 2
CHUNK = SEQ_PER_CHUNK * SEQ
N_CHUNKS = SEQ_PER_W // SEQ_PER_CHUNK
IDX_SUB = 80
N_SUB = CHUNK // IDX_SUB
GROUPS = CHUNK // LANES
NACC = 8
BIAS_ROWS = 104
EPS = 1e-12


def _rsqrt_newton(x):
    i = plsc.bitcast(x, jnp.int32)
    i = jnp.int32(0x5F3759DF) - lax.shift_right_logical(i, 1)
    y = plsc.bitcast(i, jnp.float32)
    for _ in range(3):
        y = y * (1.5 - 0.5 * x * y * y)
    return y


def _sc_kernel(ids_hbm, word_hbm, type_hbm, pos_hbm, gamma_hbm, beta_hbm,
               out_hbm, idx_v, pair_v, rows_v, out_v, bias_v, type_s,
               gamma_v, beta_v, sem):
    wid = lax.axis_index("s") * NC + lax.axis_index("c")
    iota = lax.iota(jnp.int32, LANES)

    pltpu.sync_copy(pos_hbm.at[pl.ds(0, BIAS_ROWS)], bias_v)
    pltpu.sync_copy(type_hbm, type_s)
    pltpu.sync_copy(gamma_hbm, gamma_v)
    pltpu.sync_copy(beta_hbm, beta_v)

    @plsc.parallel_loop(0, SEQ // 2, 1, unroll=2)
    def add_t0(r):
        for c in range(2 * EMB // LANES):
            sl = pl.ds(c * LANES, LANES)
            tsl = pl.ds((c % (EMB // LANES)) * LANES, LANES)
            bias_v[r, sl] = bias_v[r, sl] + type_s[0, tsl]

    seq_base = wid * SEQ_PER_W

    def chunk_body(ci, carry):
        seq0 = seq_base + ci * SEQ_PER_CHUNK
        tok0 = seq0 * SEQ
        pltpu.sync_copy(ids_hbm.at[pl.ds(tok0, CHUNK)], idx_v)

        @plsc.parallel_loop(0, CHUNK // LANES, 1, unroll=2)
        def shift_ids(i):
            sl = pl.ds(i * LANES, LANES)
            pair_v[sl] = lax.shift_right_logical(idx_v[sl], 1)

        copies = [
            pltpu.async_copy(
                word_hbm.at[pair_v.at[pl.ds(k * IDX_SUB, IDX_SUB)]],
                rows_v.at[pl.ds(k * IDX_SUB, IDX_SUB)],
                sem,
            )
            for k in range(N_SUB)
        ]
        for c in copies:
            c.wait()

        @plsc.parallel_loop(0, GROUPS, 1, unroll=1)
        def group_body(g):
            tok_vec = g * LANES + iota
            pos_vec = lax.rem(tok_vec, SEQ)
            ids_vec = idx_v[pl.ds(g * LANES, LANES)]
            half = lax.shift_left(
                lax.bitwise_and(ids_vec, jnp.int32(1)), 6)
            brow = lax.shift_right_logical(pos_vec, 1)
            bhalf = lax.shift_left(
                lax.bitwise_and(pos_vec, jnp.int32(1)), 6)
            accs = [jnp.zeros((LANES,), jnp.float32) for _ in range(NACC)]
            acc2s = [jnp.zeros((LANES,), jnp.float32) for _ in range(NACC)]
            for j in range(EMB):
                dcol = lax.bitwise_and(j + iota, jnp.int32(EMB - 1))
                v = plsc.load_gather(rows_v, [tok_vec, half + dcol])
                b = plsc.load_gather(bias_v, [brow, bhalf + dcol])
                s = v + b
                k = j % NACC
                accs[k] = accs[k] + s
                acc2s[k] = acc2s[k] + s * s
            while len(accs) > 1:
                accs = [a + b for a, b in zip(accs[::2], accs[1::2])]
                acc2s = [a + b for a, b in zip(acc2s[::2], acc2s[1::2])]
            mean = accs[0] * (1.0 / EMB)
            var = acc2s[0] * (1.0 / EMB) - mean * mean
            rstd = _rsqrt_newton(var + EPS)
            gvecs = [gamma_v[pl.ds(c * LANES, LANES)] for c in range(EMB // LANES)]
            bvecs = [beta_v[pl.ds(c * LANES, LANES)] for c in range(EMB // LANES)]
            for r in range(LANES):
                m_b = mean[r]
                rs_b = rstd[r]
                hoff = half[r]
                tok_r = g * LANES + r
                pos_r = lax.rem(tok_r, SEQ)
                boff = lax.shift_left(lax.bitwise_and(pos_r, jnp.int32(1)), 6)
                brow_r = lax.shift_right_logical(pos_r, 1)
                for c in range(EMB // LANES):
                    sl = pl.ds(c * LANES, LANES)
                    s = (rows_v[tok_r, pl.ds(hoff + c * LANES, LANES)]
                         + bias_v[brow_r, pl.ds(boff + c * LANES, LANES)])
                    y = (s - m_b) * rs_b * gvecs[c] + bvecs[c]
                    out_v[tok_r, sl] = y
            return None

        for q in range(SEQ_PER_CHUNK):
            pltpu.sync_copy(out_v.at[pl.ds(q * SEQ, SEQ)], out_hbm.at[seq0 + q])
        return carry

    lax.fori_loop(0, N_CHUNKS, chunk_body, 0)


def kernel(input_ids, word_emb, type_emb, pos_emb, gamma, beta):
    batch, seq = input_ids.shape
    ids1d = input_ids.reshape(batch * seq).astype(jnp.int32)
    word2 = word_emb.reshape(word_emb.shape[0] // 2, 2 * EMB)
    pos2 = pos_emb.reshape(pos_emb.shape[0] // 2, 2 * EMB)
    mesh = plsc.VectorSubcoreMesh(core_axis_name="c", subcore_axis_name="s")
    run = pl.kernel(
        _sc_kernel,
        out_type=jax.ShapeDtypeStruct((BATCH, SEQ, EMB), jnp.float32),
        mesh=mesh,
        compiler_params=pltpu.CompilerParams(needs_layout_passes=False),
        scratch_types=[
            pltpu.VMEM((CHUNK,), jnp.int32),
            pltpu.VMEM((CHUNK,), jnp.int32),
            pltpu.VMEM((CHUNK, 2 * EMB), jnp.float32),
            pltpu.VMEM((CHUNK, EMB), jnp.float32),
            pltpu.VMEM((BIAS_ROWS, 2 * EMB), jnp.float32),
            pltpu.VMEM((2, EMB), jnp.float32),
            pltpu.VMEM((EMB,), jnp.float32),
            pltpu.VMEM((EMB,), jnp.float32),
            pltpu.SemaphoreType.DMA,
        ],
    )
    return run(ids1d, word2, type_emb, pos2, gamma, beta)


# speedup vs baseline: 1.9705x; 1.0095x over previous
import functools

import jax
import jax.numpy as jnp
from jax import lax
from jax.experimental import pallas as pl
from jax.experimental.pallas import tpu as pltpu
from jax.experimental.pallas import tpu_sc as plsc

EMB = 64
SEQ = 200
BATCH = 1024
NC = 2
NS = 16
NW = NC * NS
LANES = 16
SEQ_PER_W = BATCH // NW
SEQ_PER_CHUNK = 2
CHUNK = SEQ_PER_CHUNK * SEQ
N_CHUNKS = SEQ_PER_W // SEQ_PER_CHUNK
IDX_SUB = 80
N_SUB = CHUNK // IDX_SUB
GROUPS = CHUNK // LANES
NACC = 8
BIAS_ROWS = 104
EPS = 1e-12


def _rsqrt_newton(x):
    i = plsc.bitcast(x, jnp.int32)
    i = jnp.int32(0x5F3759DF) - lax.shift_right_logical(i, 1)
    y = plsc.bitcast(i, jnp.float32)
    for _ in range(3):
        y = y * (1.5 - 0.5 * x * y * y)
    return y


def _sc_kernel(ids_hbm, word_hbm, type_hbm, pos_hbm, gamma_hbm, beta_hbm,
               out_hbm, idx_v, pair_v, rows_v, out_v, bias_v, type_s,
               gamma_v, beta_v, sem):
    wid = lax.axis_index("s") * NC + lax.axis_index("c")
    iota = lax.iota(jnp.int32, LANES)

    pltpu.sync_copy(pos_hbm.at[pl.ds(0, BIAS_ROWS)], bias_v)
    pltpu.sync_copy(type_hbm, type_s)
    pltpu.sync_copy(gamma_hbm, gamma_v)
    pltpu.sync_copy(beta_hbm, beta_v)

    @plsc.parallel_loop(0, SEQ // 2, 1, unroll=2)
    def add_t0(r):
        for c in range(2 * EMB // LANES):
            sl = pl.ds(c * LANES, LANES)
            tsl = pl.ds((c % (EMB // LANES)) * LANES, LANES)
            bias_v[r, sl] = bias_v[r, sl] + type_s[0, tsl]

    seq_base = wid * SEQ_PER_W

    def chunk_body(ci, carry):
        seq0 = seq_base + ci * SEQ_PER_CHUNK
        tok0 = seq0 * SEQ
        pltpu.sync_copy(ids_hbm.at[pl.ds(tok0, CHUNK)], idx_v)

        @plsc.parallel_loop(0, CHUNK // LANES, 1, unroll=2)
        def shift_ids(i):
            sl = pl.ds(i * LANES, LANES)
            pair_v[sl] = lax.shift_right_logical(idx_v[sl], 1)

        copies = [
            pltpu.async_copy(
                word_hbm.at[pair_v.at[pl.ds(k * IDX_SUB, IDX_SUB)]],
                rows_v.at[pl.ds(k * IDX_SUB, IDX_SUB)],
                sem,
            )
            for k in range(N_SUB)
        ]

        def group_body(g):
            tok_vec = g * LANES + iota
            pos_vec = lax.rem(tok_vec, SEQ)
            ids_vec = idx_v[pl.ds(g * LANES, LANES)]
            half = lax.shift_left(
                lax.bitwise_and(ids_vec, jnp.int32(1)), 6)
            brow = lax.shift_right_logical(pos_vec, 1)
            bhalf = lax.shift_left(
                lax.bitwise_and(pos_vec, jnp.int32(1)), 6)
            accs = [jnp.zeros((LANES,), jnp.float32) for _ in range(NACC)]
            acc2s = [jnp.zeros((LANES,), jnp.float32) for _ in range(NACC)]
            for j in range(EMB):
                dcol = lax.bitwise_and(j + iota, jnp.int32(EMB - 1))
                v = plsc.load_gather(rows_v, [tok_vec, half + dcol])
                b = plsc.load_gather(bias_v, [brow, bhalf + dcol])
                s = v + b
                k = j % NACC
                accs[k] = accs[k] + s
                acc2s[k] = acc2s[k] + s * s
            while len(accs) > 1:
                accs = [a + b for a, b in zip(accs[::2], accs[1::2])]
                acc2s = [a + b for a, b in zip(acc2s[::2], acc2s[1::2])]
            mean = accs[0] * (1.0 / EMB)
            var = acc2s[0] * (1.0 / EMB) - mean * mean
            rstd = _rsqrt_newton(var + EPS)
            gvecs = [gamma_v[pl.ds(c * LANES, LANES)] for c in range(EMB // LANES)]
            bvecs = [beta_v[pl.ds(c * LANES, LANES)] for c in range(EMB // LANES)]
            for r in range(LANES):
                m_b = mean[r]
                rs_b = rstd[r]
                hoff = half[r]
                tok_r = g * LANES + r
                pos_r = lax.rem(tok_r, SEQ)
                boff = lax.shift_left(lax.bitwise_and(pos_r, jnp.int32(1)), 6)
                brow_r = lax.shift_right_logical(pos_r, 1)
                for c in range(EMB // LANES):
                    sl = pl.ds(c * LANES, LANES)
                    s = (rows_v[tok_r, pl.ds(hoff + c * LANES, LANES)]
                         + bias_v[brow_r, pl.ds(boff + c * LANES, LANES)])
                    y = (s - m_b) * rs_b * gvecs[c] + bvecs[c]
                    out_v[tok_r, sl] = y
            return None

        gp = GROUPS // N_SUB
        for k in range(N_SUB):
            copies[k].wait()

            @plsc.parallel_loop(0, gp, 1, unroll=1)
            def sub_body(g0, k=k):
                group_body(k * gp + g0)

        for q in range(SEQ_PER_CHUNK):
            pltpu.sync_copy(out_v.at[pl.ds(q * SEQ, SEQ)], out_hbm.at[seq0 + q])
        return carry

    lax.fori_loop(0, N_CHUNKS, chunk_body, 0)


def kernel(input_ids, word_emb, type_emb, pos_emb, gamma, beta):
    batch, seq = input_ids.shape
    ids1d = input_ids.reshape(batch * seq).astype(jnp.int32)
    word2 = word_emb.reshape(word_emb.shape[0] // 2, 2 * EMB)
    pos2 = pos_emb.reshape(pos_emb.shape[0] // 2, 2 * EMB)
    mesh = plsc.VectorSubcoreMesh(core_axis_name="c", subcore_axis_name="s")
    run = pl.kernel(
        _sc_kernel,
        out_type=jax.ShapeDtypeStruct((BATCH, SEQ, EMB), jnp.float32),
        mesh=mesh,
        compiler_params=pltpu.CompilerParams(needs_layout_passes=False),
        scratch_types=[
            pltpu.VMEM((CHUNK,), jnp.int32),
            pltpu.VMEM((CHUNK,), jnp.int32),
            pltpu.VMEM((CHUNK, 2 * EMB), jnp.float32),
            pltpu.VMEM((CHUNK, EMB), jnp.float32),
            pltpu.VMEM((BIAS_ROWS, 2 * EMB), jnp.float32),
            pltpu.VMEM((2, EMB), jnp.float32),
            pltpu.VMEM((EMB,), jnp.float32),
            pltpu.VMEM((EMB,), jnp.float32),
            pltpu.SemaphoreType.DMA,
        ],
    )
    return run(ids1d, word2, type_emb, pos2, gamma, beta)
